# baseline (speedup 1.0000x reference)
import functools

import jax
import jax.numpy as jnp
from jax import lax
from jax.experimental import pallas as pl
from jax.experimental.pallas import tpu as pltpu
from jax.experimental.pallas import tpu_sc as plsc

D = 768
L = 16
NJ = D // L
C = 8


def _bcast_total(v):
    lanes = lax.iota(jnp.int32, L)
    for k in (1, 2, 4, 8):
        perm = lanes ^ k
        v = v + v.at[perm].get(mode="promise_in_bounds", unique_indices=True)
    return v


def _rsqrt(x):
    i = plsc.bitcast(x, jnp.int32)
    i = jnp.int32(0x5F3759DF) - lax.shift_right_logical(i, 1)
    y = plsc.bitcast(i, jnp.float32)
    for _ in range(2):
        y = y * (1.5 - 0.5 * x * y * y)
    return y


def kernel(x, seg, tok_embed, pos_embed, seg_embed, ln_gamma, ln_beta):
    B, S = x.shape
    N = B * S

    info = plsc.get_sparse_core_info()
    NC, NS = info.num_cores, info.num_subcores
    NW = NC * NS
    n_per_w = N // NW
    BW = B // NW
    PC = S // C
    K = 16
    n_chunks = n_per_w // K
    cpb = (BW * C) // K
    NBUF = 2

    xr = (x.astype(jnp.int32).reshape(NW, BW, PC, C)
          .transpose(0, 2, 1, 3).reshape(N))
    sr = (seg.astype(jnp.int32).reshape(NW, BW, PC, C)
          .transpose(0, 2, 1, 3).reshape(N))
    posseg = (seg_embed[:, None, :] + pos_embed[None, :, :]).reshape(2 * S, D)

    mesh = plsc.VectorSubcoreMesh(core_axis_name="c", subcore_axis_name="s")

    @functools.partial(
        pl.kernel,
        mesh=mesh,
        compiler_params=pltpu.CompilerParams(needs_layout_passes=False),
        out_type=jax.ShapeDtypeStruct((N, D), jnp.float32),
        scratch_types=[
            pltpu.VMEM((n_per_w,), jnp.int32),
            pltpu.VMEM((n_per_w,), jnp.int32),
            pltpu.VMEM_SHARED((NS * n_per_w,), jnp.int32),
            pltpu.VMEM((NBUF, K, D), jnp.float32),
            pltpu.VMEM((2 * C, D), jnp.float32),
            pltpu.VMEM((K, D), jnp.float32),
            pltpu.VMEM((2, K, L), jnp.float32),
            pltpu.VMEM((NBUF, K, D), jnp.float32),
            pltpu.VMEM((D,), jnp.float32),
            pltpu.VMEM((D,), jnp.float32),
            pltpu.SMEM((NBUF, K), jnp.int32),
            pltpu.SemaphoreType.DMA((NBUF,)),
            pltpu.SemaphoreType.DMA((NBUF,)),
            pltpu.SemaphoreType.DMA((NBUF,)),
        ],
    )
    def sc_embed(tok_hbm, ps_hbm, idxt_hbm, segr_hbm, g_hbm, b_hbm, out_hbm,
                 idxt_v, segr_v, seg_sh, buf_tok, pos2_v, buf_v, buf_mr, buf_o,
                 g_v, b_v, seg_sm, sem_g, sem_o, sem_s):
        sid = lax.axis_index("s")
        wid = sid * NC + lax.axis_index("c")
        base = wid * n_per_w
        sbase = sid * n_per_w
        bbase = wid * BW
        pltpu.sync_copy(idxt_hbm.at[pl.ds(base, n_per_w)], idxt_v)
        pltpu.sync_copy(g_hbm, g_v)
        pltpu.sync_copy(b_hbm, b_v)
        pltpu.sync_copy(segr_hbm.at[pl.ds(base, n_per_w)], segr_v)
        pltpu.sync_copy(segr_v, seg_sh.at[pl.ds(sbase, n_per_w)])

        def issue_gather(c, slot):
            pltpu.async_copy(tok_hbm.at[idxt_v.at[pl.ds(c * K, K)]],
                             buf_tok.at[slot], sem_g.at[slot])
            pltpu.async_copy(seg_sh.at[pl.ds(sbase + c * K, K)],
                             seg_sm.at[slot], sem_s.at[slot])

        def wait_gather(c, slot):
            pltpu.make_async_copy(tok_hbm.at[idxt_v.at[pl.ds(c * K, K)]],
                                  buf_tok.at[slot], sem_g.at[slot]).wait()
            pltpu.make_async_copy(seg_sh.at[pl.ds(sbase + c * K, K)],
                                  seg_sm.at[slot], sem_s.at[slot]).wait()

        def stage_pos_block(pc):
            pltpu.sync_copy(ps_hbm.at[pl.ds(pc * C, C)],
                            pos2_v.at[pl.ds(0, C)])
            pltpu.sync_copy(ps_hbm.at[pl.ds(S + pc * C, C)],
                            pos2_v.at[pl.ds(C, C)])

        def compute_chunk(c, slot):
            bt = buf_tok.at[slot]
            bo = buf_o.at[slot]

            def stage_a(t):
                seg_t = seg_sm[slot, t]
                s_local = lax.rem(t, C)
                row = seg_t * C + s_local
                z = jnp.zeros((L,), jnp.float32)
                s1 = [z, z, z, z]
                s2 = [z, z, z, z]
                for j in range(NJ):
                    p = pos2_v[row, pl.ds(j * L, L)]
                    v = bt[t, pl.ds(j * L, L)] + p
                    buf_v[t, pl.ds(j * L, L)] = v
                    s1[j % 4] = s1[j % 4] + v
                    s2[j % 4] = s2[j % 4] + v * v
                return (s1[0] + s1[1]) + (s1[2] + s1[3]), \
                       (s2[0] + s2[1]) + (s2[2] + s2[3])

            def stage_b(t, sa, sb):
                tot1 = _bcast_total(sa)
                tot2 = _bcast_total(sb)
                mean = tot1 * (1.0 / D)
                var = tot2 * (1.0 / D) - mean * mean
                r = _rsqrt(var + 1e-5)
                buf_mr[0, t] = mean
                buf_mr[1, t] = r

            sa0, sb0 = stage_a(0)

            def token_body(t, carry):
                sa, sb = carry
                nsa, nsb = stage_a(t + 1)
                stage_b(t, sa, sb)
                return nsa, nsb

            sa_l, sb_l = plsc.parallel_loop(0, K - 1, 1, unroll=1,
                                            carry=(sa0, sb0))(token_body)
            stage_b(K - 1, sa_l, sb_l)

            for half in range(2):
                t0 = half * (K // 2)
                ms = [buf_mr[0, t0 + t] for t in range(K // 2)]
                rs = [buf_mr[1, t0 + t] for t in range(K // 2)]

                def gb_body(j, t0=t0, ms=ms, rs=rs):
                    gj = g_v[pl.ds(j * L, L)]
                    bj = b_v[pl.ds(j * L, L)]
                    for t in range(K // 2):
                        o = (buf_v[t0 + t, pl.ds(j * L, L)] - ms[t]) * rs[t]
                        bo[t0 + t, pl.ds(j * L, L)] = o * gj + bj

                plsc.parallel_loop(0, NJ, 1, unroll=2)(gb_body)

        def out_rows(c):
            pc = c // cpb
            r = lax.rem(c, cpb)
            return (bbase + 2 * r) * S + pc * C, (bbase + 2 * r + 1) * S + pc * C

        def issue_out(c, slot):
            r0, r1 = out_rows(c)
            pltpu.async_copy(buf_o.at[slot].at[pl.ds(0, C)],
                             out_hbm.at[pl.ds(r0, C)], sem_o.at[slot])
            pltpu.async_copy(buf_o.at[slot].at[pl.ds(C, C)],
                             out_hbm.at[pl.ds(r1, C)], sem_o.at[slot])

        def wait_out(c, slot):
            r0, r1 = out_rows(c)
            pltpu.make_async_copy(buf_o.at[slot].at[pl.ds(0, C)],
                                  out_hbm.at[pl.ds(r0, C)], sem_o.at[slot]).wait()
            pltpu.make_async_copy(buf_o.at[slot].at[pl.ds(C, C)],
                                  out_hbm.at[pl.ds(r1, C)], sem_o.at[slot]).wait()

        stage_pos_block(0)
        issue_gather(0, 0)

        def group_body(g, _):
            for b in range(NBUF):
                c = g * NBUF + b
                nxt = (b + 1) % NBUF
                @pl.when(c >= 1)
                def _():
                    wait_out(c - 1, nxt)

                @pl.when(c + 1 < n_chunks)
                def _():
                    issue_gather(c + 1, nxt)

                @pl.when(jnp.logical_and(lax.rem(c, cpb) == 0, c > 0))
                def _():
                    stage_pos_block(c // cpb)

                wait_gather(c, b)
                compute_chunk(c, b)
                issue_out(c, b)
            return 0

        lax.fori_loop(0, n_chunks // NBUF, group_body, 0, unroll=False)
        wait_out(n_chunks - 1, (n_chunks - 1) % NBUF)

    out = sc_embed(tok_embed, posseg, xr, sr, ln_gamma, ln_beta)
    return out.reshape(B, S, D)


# speedup vs baseline: 1.4388x; 1.4388x over previous
import functools

import jax
import jax.numpy as jnp
from jax import lax
from jax.experimental import pallas as pl
from jax.experimental.pallas import tpu as pltpu
from jax.experimental.pallas import tpu_sc as plsc

D = 768
L = 16
NJ = D // L
C = 8


def _bcast_total(v):
    lanes = lax.iota(jnp.int32, L)
    for k in (1, 2, 4, 8):
        perm = lanes ^ k
        v = v + v.at[perm].get(mode="promise_in_bounds", unique_indices=True)
    return v


def _rsqrt(x):
    i = plsc.bitcast(x, jnp.int32)
    i = jnp.int32(0x5F3759DF) - lax.shift_right_logical(i, 1)
    y = plsc.bitcast(i, jnp.float32)
    for _ in range(2):
        y = y * (1.5 - 0.5 * x * y * y)
    return y


def kernel(x, seg, tok_embed, pos_embed, seg_embed, ln_gamma, ln_beta):
    B, S = x.shape
    N = B * S

    info = plsc.get_sparse_core_info()
    NC, NS = info.num_cores, info.num_subcores
    NW = NC * NS
    n_per_w = N // NW
    BW = B // NW
    PC = S // C
    K = 16
    n_chunks = n_per_w // K
    cpb = (BW * C) // K
    NBUF = 2

    xr = (x.astype(jnp.int32).reshape(NW, BW, PC, C)
          .transpose(0, 2, 1, 3).reshape(N))
    sr = (seg.astype(jnp.int32).reshape(NW, BW, PC, C)
          .transpose(0, 2, 1, 3).reshape(N))
    posseg = (seg_embed[:, None, :] + pos_embed[None, :, :]).reshape(2 * S, D)

    mesh = plsc.VectorSubcoreMesh(core_axis_name="c", subcore_axis_name="s")

    @functools.partial(
        pl.kernel,
        mesh=mesh,
        compiler_params=pltpu.CompilerParams(needs_layout_passes=False),
        out_type=jax.ShapeDtypeStruct((N, D), jnp.float32),
        scratch_types=[
            pltpu.VMEM((n_per_w,), jnp.int32),
            pltpu.VMEM((n_per_w,), jnp.int32),
            pltpu.VMEM((NBUF, K, D), jnp.float32),
            pltpu.VMEM((2 * C, D), jnp.float32),
            pltpu.VMEM((K, D), jnp.float32),
            pltpu.VMEM((K, D), jnp.float32),
            pltpu.VMEM((2, K, L), jnp.float32),
            pltpu.VMEM((NBUF, K, D), jnp.float32),
            pltpu.VMEM((D,), jnp.float32),
            pltpu.VMEM((D,), jnp.float32),
            pltpu.SemaphoreType.DMA((NBUF,)),
            pltpu.SemaphoreType.DMA((NBUF,)),
        ],
    )
    def sc_embed(tok_hbm, ps_hbm, idxt_hbm, segr_hbm, g_hbm, b_hbm, out_hbm,
                 idxt_v, segr_v, buf_tok, pos2_v, buf_ps, buf_v, buf_mr, buf_o,
                 g_v, b_v, sem_g, sem_o):
        wid = lax.axis_index("s") * NC + lax.axis_index("c")
        base = wid * n_per_w
        bbase = wid * BW
        pltpu.sync_copy(idxt_hbm.at[pl.ds(base, n_per_w)], idxt_v)
        pltpu.sync_copy(segr_hbm.at[pl.ds(base, n_per_w)], segr_v)
        pltpu.sync_copy(g_hbm, g_v)
        pltpu.sync_copy(b_hbm, b_v)

        def issue_gather(c, slot):
            pltpu.async_copy(tok_hbm.at[idxt_v.at[pl.ds(c * K, K)]],
                             buf_tok.at[slot], sem_g.at[slot])

        def wait_gather(c, slot):
            pltpu.make_async_copy(tok_hbm.at[idxt_v.at[pl.ds(c * K, K)]],
                                  buf_tok.at[slot], sem_g.at[slot]).wait()

        def stage_pos_block(pc):
            pltpu.sync_copy(ps_hbm.at[pl.ds(pc * C, C)],
                            pos2_v.at[pl.ds(0, C)])
            pltpu.sync_copy(ps_hbm.at[pl.ds(S + pc * C, C)],
                            pos2_v.at[pl.ds(C, C)])

        def compute_chunk(c, slot):
            bt = buf_tok.at[slot]
            bo = buf_o.at[slot]

            for half in range(2):
                t0 = half * (K // 2)
                msks = [
                    plsc.load_gather(
                        segr_v,
                        [c * K + (t0 + t) + jnp.zeros((L,), jnp.int32)]) != 0
                    for t in range(K // 2)
                ]

                def sel_body(j, t0=t0, msks=msks):
                    for t in range(K // 2):
                        p0 = pos2_v[t, pl.ds(j * L, L)]
                        p1 = pos2_v[C + t, pl.ds(j * L, L)]
                        buf_ps[t0 + t, pl.ds(j * L, L)] = jnp.where(
                            msks[t], p1, p0)

                plsc.parallel_loop(0, NJ, 1, unroll=2)(sel_body)
            wait_gather(c, slot)

            def stage_a(t):
                z = jnp.zeros((L,), jnp.float32)
                s1 = [z, z, z, z]
                s2 = [z, z, z, z]
                for j in range(NJ):
                    v = bt[t, pl.ds(j * L, L)] + buf_ps[t, pl.ds(j * L, L)]
                    buf_v[t, pl.ds(j * L, L)] = v
                    s1[j % 4] = s1[j % 4] + v
                    s2[j % 4] = s2[j % 4] + v * v
                return (s1[0] + s1[1]) + (s1[2] + s1[3]), \
                       (s2[0] + s2[1]) + (s2[2] + s2[3])

            def stage_b(t, sa, sb):
                tot1 = _bcast_total(sa)
                tot2 = _bcast_total(sb)
                mean = tot1 * (1.0 / D)
                var = tot2 * (1.0 / D) - mean * mean
                r = _rsqrt(var + 1e-5)
                buf_mr[0, t] = mean
                buf_mr[1, t] = r

            sa0, sb0 = stage_a(0)

            def token_body(t, carry):
                sa, sb = carry
                nsa, nsb = stage_a(t + 1)
                stage_b(t, sa, sb)
                return nsa, nsb

            sa_l, sb_l = lax.fori_loop(0, K - 1, token_body, (sa0, sb0),
                                       unroll=False)
            stage_b(K - 1, sa_l, sb_l)

            for half in range(2):
                t0 = half * (K // 2)
                ms = [buf_mr[0, t0 + t] for t in range(K // 2)]
                rs = [buf_mr[1, t0 + t] for t in range(K // 2)]

                def gb_body(j, t0=t0, ms=ms, rs=rs):
                    gj = g_v[pl.ds(j * L, L)]
                    bj = b_v[pl.ds(j * L, L)]
                    for t in range(K // 2):
                        o = (buf_v[t0 + t, pl.ds(j * L, L)] - ms[t]) * rs[t]
                        bo[t0 + t, pl.ds(j * L, L)] = o * gj + bj

                plsc.parallel_loop(0, NJ, 1, unroll=2)(gb_body)

        def out_rows(c):
            pc = c // cpb
            r = lax.rem(c, cpb)
            return (bbase + 2 * r) * S + pc * C, (bbase + 2 * r + 1) * S + pc * C

        def issue_out(c, slot):
            r0, r1 = out_rows(c)
            pltpu.async_copy(buf_o.at[slot].at[pl.ds(0, C)],
                             out_hbm.at[pl.ds(r0, C)], sem_o.at[slot])
            pltpu.async_copy(buf_o.at[slot].at[pl.ds(C, C)],
                             out_hbm.at[pl.ds(r1, C)], sem_o.at[slot])

        def wait_out(c, slot):
            r0, r1 = out_rows(c)
            pltpu.make_async_copy(buf_o.at[slot].at[pl.ds(0, C)],
                                  out_hbm.at[pl.ds(r0, C)], sem_o.at[slot]).wait()
            pltpu.make_async_copy(buf_o.at[slot].at[pl.ds(C, C)],
                                  out_hbm.at[pl.ds(r1, C)], sem_o.at[slot]).wait()

        stage_pos_block(0)
        issue_gather(0, 0)

        def group_body(g, _):
            for b in range(NBUF):
                c = g * NBUF + b
                nxt = (b + 1) % NBUF
                @pl.when(c >= 1)
                def _():
                    wait_out(c - 1, nxt)

                @pl.when(c + 1 < n_chunks)
                def _():
                    issue_gather(c + 1, nxt)

                @pl.when(jnp.logical_and(lax.rem(c, cpb) == 0, c > 0))
                def _():
                    stage_pos_block(c // cpb)

                compute_chunk(c, b)
                issue_out(c, b)
            return 0

        lax.fori_loop(0, n_chunks // NBUF, group_body, 0, unroll=False)
        wait_out(n_chunks - 1, (n_chunks - 1) % NBUF)

    out = sc_embed(tok_embed, posseg, xr, sr, ln_gamma, ln_beta)
    return out.reshape(B, S, D)


# speedup vs baseline: 1.6806x; 1.1681x over previous
import functools

import jax
import jax.numpy as jnp
from jax import lax
from jax.experimental import pallas as pl
from jax.experimental.pallas import tpu as pltpu
from jax.experimental.pallas import tpu_sc as plsc

D = 768
L = 16
NJ = D // L
C = 8


def _bcast_total(v):
    lanes = lax.iota(jnp.int32, L)
    for k in (1, 2, 4, 8):
        perm = lanes ^ k
        v = v + v.at[perm].get(mode="promise_in_bounds", unique_indices=True)
    return v


def _rsqrt(x):
    i = plsc.bitcast(x, jnp.int32)
    i = jnp.int32(0x5F3759DF) - lax.shift_right_logical(i, 1)
    y = plsc.bitcast(i, jnp.float32)
    for _ in range(2):
        y = y * (1.5 - 0.5 * x * y * y)
    return y


def kernel(x, seg, tok_embed, pos_embed, seg_embed, ln_gamma, ln_beta):
    B, S = x.shape
    N = B * S

    info = plsc.get_sparse_core_info()
    NC, NS = info.num_cores, info.num_subcores
    NW = NC * NS
    n_per_w = N // NW
    BW = B // NW
    PC = S // C
    K = 16
    n_chunks = n_per_w // K
    cpb = (BW * C) // K
    NBUF = 4

    xr = (x.astype(jnp.int32).reshape(NW, BW, PC, C)
          .transpose(0, 2, 1, 3).reshape(N))
    sr = (seg.astype(jnp.int32).reshape(NW, BW, PC, C)
          .transpose(0, 2, 1, 3).reshape(N))
    posseg = (seg_embed[:, None, :] + pos_embed[None, :, :]).reshape(2 * S, D)

    mesh = plsc.VectorSubcoreMesh(core_axis_name="c", subcore_axis_name="s")

    @functools.partial(
        pl.kernel,
        mesh=mesh,
        compiler_params=pltpu.CompilerParams(needs_layout_passes=False),
        out_type=jax.ShapeDtypeStruct((N, D), jnp.float32),
        scratch_types=[
            pltpu.VMEM((n_per_w,), jnp.int32),
            pltpu.VMEM((n_per_w,), jnp.int32),
            pltpu.VMEM((NBUF, K, D), jnp.float32),
            pltpu.VMEM((2 * C, D), jnp.float32),
            pltpu.VMEM((K, D), jnp.float32),
            pltpu.VMEM((K, D), jnp.float32),
            pltpu.VMEM((2, K, L), jnp.float32),
            pltpu.VMEM((D,), jnp.float32),
            pltpu.VMEM((D,), jnp.float32),
            pltpu.SemaphoreType.DMA((NBUF,)),
            pltpu.SemaphoreType.DMA((NBUF,)),
        ],
    )
    def sc_embed(tok_hbm, ps_hbm, idxt_hbm, segr_hbm, g_hbm, b_hbm, out_hbm,
                 idxt_v, segr_v, buf_tok, pos2_v, buf_ps, buf_v, buf_mr,
                 g_v, b_v, sem_g, sem_o):
        wid = lax.axis_index("s") * NC + lax.axis_index("c")
        base = wid * n_per_w
        bbase = wid * BW
        pltpu.sync_copy(idxt_hbm.at[pl.ds(base, n_per_w)], idxt_v)
        pltpu.sync_copy(segr_hbm.at[pl.ds(base, n_per_w)], segr_v)
        pltpu.sync_copy(g_hbm, g_v)
        pltpu.sync_copy(b_hbm, b_v)

        def issue_gather(c, slot):
            pltpu.async_copy(tok_hbm.at[idxt_v.at[pl.ds(c * K, K)]],
                             buf_tok.at[slot], sem_g.at[slot])

        def wait_gather(c, slot):
            pltpu.make_async_copy(tok_hbm.at[idxt_v.at[pl.ds(c * K, K)]],
                                  buf_tok.at[slot], sem_g.at[slot]).wait()

        def stage_pos_block(pc):
            pltpu.sync_copy(ps_hbm.at[pl.ds(pc * C, C)],
                            pos2_v.at[pl.ds(0, C)])
            pltpu.sync_copy(ps_hbm.at[pl.ds(S + pc * C, C)],
                            pos2_v.at[pl.ds(C, C)])

        def compute_chunk(c, slot):
            bt = buf_tok.at[slot]
            bo = buf_tok.at[slot]

            for half in range(2):
                t0 = half * (K // 2)
                msks = [
                    plsc.load_gather(
                        segr_v,
                        [c * K + (t0 + t) + jnp.zeros((L,), jnp.int32)]) != 0
                    for t in range(K // 2)
                ]

                def sel_body(j, t0=t0, msks=msks):
                    for t in range(K // 2):
                        p0 = pos2_v[t, pl.ds(j * L, L)]
                        p1 = pos2_v[C + t, pl.ds(j * L, L)]
                        buf_ps[t0 + t, pl.ds(j * L, L)] = jnp.where(
                            msks[t], p1, p0)

                plsc.parallel_loop(0, NJ, 1, unroll=2)(sel_body)
            wait_gather(c, slot)

            def stage_a(t):
                z = jnp.zeros((L,), jnp.float32)
                s1 = [z, z, z, z]
                s2 = [z, z, z, z]
                for j in range(NJ):
                    v = bt[t, pl.ds(j * L, L)] + buf_ps[t, pl.ds(j * L, L)]
                    buf_v[t, pl.ds(j * L, L)] = v
                    s1[j % 4] = s1[j % 4] + v
                    s2[j % 4] = s2[j % 4] + v * v
                return (s1[0] + s1[1]) + (s1[2] + s1[3]), \
                       (s2[0] + s2[1]) + (s2[2] + s2[3])

            def stage_b(t, sa, sb):
                tot1 = _bcast_total(sa)
                tot2 = _bcast_total(sb)
                mean = tot1 * (1.0 / D)
                var = tot2 * (1.0 / D) - mean * mean
                r = _rsqrt(var + 1e-5)
                buf_mr[0, t] = mean
                buf_mr[1, t] = r

            sa0, sb0 = stage_a(0)

            def token_body(t, carry):
                sa, sb = carry
                nsa, nsb = stage_a(t + 1)
                stage_b(t, sa, sb)
                return nsa, nsb

            sa_l, sb_l = lax.fori_loop(0, K - 1, token_body, (sa0, sb0),
                                       unroll=False)
            stage_b(K - 1, sa_l, sb_l)

            for half in range(2):
                t0 = half * (K // 2)
                ms = [buf_mr[0, t0 + t] for t in range(K // 2)]
                rs = [buf_mr[1, t0 + t] for t in range(K // 2)]

                def gb_body(j, t0=t0, ms=ms, rs=rs):
                    gj = g_v[pl.ds(j * L, L)]
                    bj = b_v[pl.ds(j * L, L)]
                    for t in range(K // 2):
                        o = (buf_v[t0 + t, pl.ds(j * L, L)] - ms[t]) * rs[t]
                        bo[t0 + t, pl.ds(j * L, L)] = o * gj + bj

                plsc.parallel_loop(0, NJ, 1, unroll=2)(gb_body)

        def out_rows(c):
            pc = c // cpb
            r = lax.rem(c, cpb)
            return (bbase + 2 * r) * S + pc * C, (bbase + 2 * r + 1) * S + pc * C

        def issue_out(c, slot):
            r0, r1 = out_rows(c)
            pltpu.async_copy(buf_tok.at[slot].at[pl.ds(0, C)],
                             out_hbm.at[pl.ds(r0, C)], sem_o.at[slot])
            pltpu.async_copy(buf_tok.at[slot].at[pl.ds(C, C)],
                             out_hbm.at[pl.ds(r1, C)], sem_o.at[slot])

        def wait_out(c, slot):
            r0, r1 = out_rows(c)
            pltpu.make_async_copy(buf_tok.at[slot].at[pl.ds(0, C)],
                                  out_hbm.at[pl.ds(r0, C)], sem_o.at[slot]).wait()
            pltpu.make_async_copy(buf_tok.at[slot].at[pl.ds(C, C)],
                                  out_hbm.at[pl.ds(r1, C)], sem_o.at[slot]).wait()

        stage_pos_block(0)
        for pre in range(NBUF - 1):
            issue_gather(pre, pre)

        def group_body(g, _):
            for b in range(NBUF):
                c = g * NBUF + b
                nxt = (b + NBUF - 1) % NBUF
                @pl.when(c >= 1)
                def _():
                    wait_out(c - 1, nxt)

                @pl.when(c + NBUF - 1 < n_chunks)
                def _():
                    issue_gather(c + NBUF - 1, nxt)

                @pl.when(jnp.logical_and(lax.rem(c, cpb) == 0, c > 0))
                def _():
                    stage_pos_block(c // cpb)

                compute_chunk(c, b)
                issue_out(c, b)
            return 0

        lax.fori_loop(0, n_chunks // NBUF, group_body, 0, unroll=False)
        wait_out(n_chunks - 1, (n_chunks - 1) % NBUF)

    out = sc_embed(tok_embed, posseg, xr, sr, ln_gamma, ln_beta)
    return out.reshape(B, S, D)
